# baseline (speedup 1.0000x reference)
import jax
import jax.numpy as jnp
from jax.experimental import pallas as pl

_N_CLASSES = 1000
_BLOCK_ROWS = 512


def _onehot_block(x_ref, o_ref):
    classes = jax.lax.broadcasted_iota(jnp.int32, (_BLOCK_ROWS, _N_CLASSES), 1)
    o_ref[...] = (x_ref[...][:, None] == classes).astype(jnp.int32)


def kernel(x):
    b, s = x.shape
    rows = b * s
    xf = x.reshape(rows)
    out = pl.pallas_call(
        _onehot_block,
        grid=(rows // _BLOCK_ROWS,),
        in_specs=[pl.BlockSpec((_BLOCK_ROWS,), lambda i: (i,))],
        out_specs=pl.BlockSpec((_BLOCK_ROWS, _N_CLASSES), lambda i: (i, 0)),
        out_shape=jax.ShapeDtypeStruct((rows, _N_CLASSES), jnp.int32),
    )(xf)
    return out.reshape(b, s, _N_CLASSES)
